# baseline (speedup 1.0000x reference)
import functools
import math

import jax
import jax.numpy as jnp
from jax import lax
from jax.experimental import pallas as pl
from jax.experimental.pallas import tpu as pltpu
from jax.experimental.pallas import tpu_sc as plsc

SCALE = 30.0
MARGIN = 0.35
K2 = SCALE / math.log(2.0)
LN2 = math.log(2.0)
W = 2048
SC_CW = 768
SC_NB = 26
TC_C0 = 79872
NEG_HUGE = -3.0e38


def _tc_stream_body(C0, B, tail_start, x_ref, tgt_ref, *rest):
    if tail_start is not None:
        xtail_ref = rest[0]
        rest = rest[1:]
    m_out, s_out, xt_out, m_scr, s_scr, xt_scr = rest

    j = pl.program_id(0)
    nj = pl.num_programs(0)

    @pl.when(j == 0)
    def _init():
        m_scr[...] = jnp.full((B, 1), -jnp.inf, jnp.float32)
        s_scr[...] = jnp.zeros((B, 1), jnp.float32)
        xt_scr[...] = jnp.zeros((B, 1), jnp.float32)

    y = x_ref[...] * K2
    lane = lax.broadcasted_iota(jnp.int32, (B, W), 1)
    tloc = tgt_ref[...] - j * W
    eq = lane == tloc
    xt_scr[...] += jnp.sum(jnp.where(eq, y, 0.0), axis=1, keepdims=True)

    def update(yv):
        m_old = m_scr[...]
        m_new = jnp.maximum(m_old, jnp.max(yv, axis=1, keepdims=True))
        p = jnp.exp2(yv - m_new)
        s_scr[...] = (
            s_scr[...] * jnp.exp2(m_old - m_new)
            + jnp.sum(p, axis=1, keepdims=True)
        )
        m_scr[...] = m_new

    if tail_start is None:
        @pl.when(j < nj - 1)
        def _full():
            update(y)

        @pl.when(j == nj - 1)
        def _tail():
            last_valid = C0 - (nj - 1) * W
            update(jnp.where(lane < last_valid, y, -jnp.inf))
    else:
        update(y)

        @pl.when(j == nj - 1)
        def _tail():
            blk0 = tail_start[0] * W
            lo, hi = tail_start[1] - blk0, tail_start[2] - blk0
            yt = xtail_ref[...] * K2
            valid = (lane >= lo) & (lane < hi)
            eq_t = (lane == tgt_ref[...] - blk0) & (lane >= lo)
            xt_scr[...] += jnp.sum(
                jnp.where(eq_t, yt, 0.0), axis=1, keepdims=True
            )
            update(jnp.where(valid, yt, -jnp.inf))

    @pl.when(j == nj - 1)
    def _emit():
        m_out[...] = m_scr[...]
        s_out[...] = s_scr[...]
        xt_out[...] = xt_scr[...]


def _tc_stream(output, tgt, C0, tail_start=None):
    B, C = output.shape
    nj = C0 // W if C0 % W == 0 else pl.cdiv(C0, W)
    shp = jax.ShapeDtypeStruct((B, 1), jnp.float32)
    in_specs = [
        pl.BlockSpec((B, W), lambda j: (0, j)),
        pl.BlockSpec((B, 1), lambda j: (0, 0)),
    ]
    args = [output, tgt]
    if tail_start is not None:
        tb = tail_start[0]
        in_specs.append(pl.BlockSpec((B, W), lambda j: (0, tb)))
        args.append(output)
    return pl.pallas_call(
        functools.partial(_tc_stream_body, C0, B, tail_start),
        grid=(nj,),
        in_specs=in_specs,
        out_specs=[pl.BlockSpec((B, 1), lambda j: (0, 0))] * 3,
        out_shape=[shp, shp, shp],
        scratch_shapes=[pltpu.VMEM((B, 1), jnp.float32)] * 3,
    )(*args)


def _combine_body(B, m_tc, s_tc, xt_tc, m_sc, s_sc, xt_sc, o_ref):
    m2 = m_sc[...] * K2
    m = jnp.maximum(m_tc[...], jnp.max(m2, axis=1, keepdims=True))
    s = (s_tc[...] * jnp.exp2(m_tc[...] - m)
         + jnp.sum(s_sc[...] * jnp.exp2(m2 - m), axis=1, keepdims=True))
    xt = xt_tc[...] + jnp.sum(xt_sc[...], axis=1, keepdims=True) * K2
    mgn = MARGIN * K2
    s_corr = s - jnp.exp2(xt - m) + jnp.exp2(xt - mgn - m)
    nll = (m + jnp.log2(s_corr) - (xt - mgn)) * LN2
    o_ref[...] = jnp.sum(nll, axis=0, keepdims=True) / B


def _combine(parts_tc, parts_sc, B):
    m_sc, s_sc, xt_sc = parts_sc
    out = pl.pallas_call(
        functools.partial(_combine_body, B),
        in_specs=[
            pl.BlockSpec((B, 1), lambda: (0, 0)),
            pl.BlockSpec((B, 1), lambda: (0, 0)),
            pl.BlockSpec((B, 1), lambda: (0, 0)),
            pl.BlockSpec((B, 16), lambda: (0, 0)),
            pl.BlockSpec((B, 16), lambda: (0, 0)),
            pl.BlockSpec((B, 16), lambda: (0, 0)),
        ],
        out_specs=pl.BlockSpec((1, 1), lambda: (0, 0)),
        out_shape=jax.ShapeDtypeStruct((1, 1), jnp.float32),
    )(*parts_tc, m_sc, s_sc, xt_sc)
    return out[0, 0]


def _sc_stream_body(C0, rows_pw, n_cores, x_hbm, tgt_hbm,
                    m_out, s_out, xt_out,
                    bufs, tgt_v, mst, sst, xtl, sem):
    wid = lax.axis_index("s") * n_cores + lax.axis_index("c")
    row0 = wid * rows_pw
    nch = SC_CW // 16

    pltpu.sync_copy(tgt_hbm.at[pl.ds(row0, rows_pw)], tgt_v)
    for r in range(rows_pw):
        mst[r, :] = jnp.full((16,), NEG_HUGE, jnp.float32)
        sst[r, :] = jnp.zeros((16,), jnp.float32)
        xtl[r, :] = jnp.zeros((16,), jnp.float32)

    def start(b, slot):
        return pltpu.async_copy(
            x_hbm.at[pl.ds(row0, rows_pw), pl.ds(C0 + b * SC_CW, SC_CW)],
            bufs.at[slot], sem)

    def drain(slot):
        pltpu.make_async_copy(
            x_hbm.at[pl.ds(row0, rows_pw), pl.ds(C0, SC_CW)],
            bufs.at[slot], sem).wait()

    iota16 = lax.iota(jnp.int32, 16)

    def consume(b, slot):
        buf = bufs.at[slot]

        def row_body(r, carry):
            cs = [buf[r, pl.ds(k * 16, 16)] for k in range(nch)]
            mx = cs[0]
            for c in cs[1:]:
                mx = jnp.maximum(mx, c)
            m_old = mst[r, :]
            m_new = jnp.maximum(m_old, mx)
            acc = sst[r, :] * jnp.exp((m_old - m_new) * SCALE)
            m30 = m_new * SCALE
            for c in cs:
                acc += jnp.exp(c * SCALE - m30)
            mst[r, :] = m_new
            sst[r, :] = acc
            return carry

        lax.fori_loop(0, rows_pw, row_body, 0)

        base = C0 + b * SC_CW
        for g in range(rows_pw // 16):
            tv = tgt_v[pl.ds(g * 16, 16)]
            local = tv - base
            for i in range(16):
                row = g * 16 + i
                li = local[i]
                ok = (li >= 0) & (li < SC_CW)
                al = pl.multiple_of(
                    jnp.clip((li // 16) * 16, 0, SC_CW - 16), 16
                )
                chunk = buf[row, pl.ds(al, 16)]
                sel = jnp.where(ok, li - al, -1)
                xtl[row, :] = jnp.where(iota16 == sel, chunk, xtl[row, :])

    start(0, 0)
    start(1, 1)

    def pair_body(g2, carry):
        b0 = g2 * 2
        drain(0)
        consume(b0, 0)

        @pl.when(b0 + 2 < SC_NB)
        def _n0():
            start(b0 + 2, 0)

        drain(1)
        consume(b0 + 1, 1)

        @pl.when(b0 + 3 < SC_NB)
        def _n1():
            start(b0 + 3, 1)

        return carry

    lax.fori_loop(0, SC_NB // 2, pair_body, 0)

    pltpu.sync_copy(mst, m_out.at[pl.ds(row0, rows_pw), :])
    pltpu.sync_copy(sst, s_out.at[pl.ds(row0, rows_pw), :])
    pltpu.sync_copy(xtl, xt_out.at[pl.ds(row0, rows_pw), :])


def _sc_stream(output, tgt_flat, C0):
    B, _ = output.shape
    info = plsc.get_sparse_core_info()
    n_workers = info.num_cores * info.num_subcores
    rows_pw = B // n_workers
    mesh = plsc.VectorSubcoreMesh(core_axis_name="c", subcore_axis_name="s")
    return pl.kernel(
        functools.partial(_sc_stream_body, C0, rows_pw, info.num_cores),
        mesh=mesh,
        out_type=[
            jax.ShapeDtypeStruct((B, 16), jnp.float32),
            jax.ShapeDtypeStruct((B, 16), jnp.float32),
            jax.ShapeDtypeStruct((B, 16), jnp.float32),
        ],
        scratch_types=[
            pltpu.VMEM((2, rows_pw, SC_CW), jnp.float32),
            pltpu.VMEM((rows_pw,), jnp.int32),
            pltpu.VMEM((rows_pw, 16), jnp.float32),
            pltpu.VMEM((rows_pw, 16), jnp.float32),
            pltpu.VMEM((rows_pw, 16), jnp.float32),
            pltpu.SemaphoreType.DMA,
        ],
    )(output, tgt_flat)


def _tc_only(output, tgt, B, C):
    parts_tc = _tc_stream(output, tgt, C)
    zeros16 = jnp.full((B, 16), NEG_HUGE, jnp.float32)
    parts_sc = (zeros16, jnp.zeros((B, 16), jnp.float32),
                jnp.zeros((B, 16), jnp.float32))
    return _combine(parts_tc, parts_sc, B)


def kernel(output, target):
    B, C = output.shape
    tgt_flat = target.astype(jnp.int32)
    tgt = tgt_flat.reshape(B, 1)

    if (B, C) != (1024, 100000):
        return _tc_only(output, tgt, B, C)

    sc_end = TC_C0 + SC_CW * SC_NB
    parts_sc = _sc_stream(output, tgt_flat, TC_C0)
    parts_tc = _tc_stream(output, tgt, TC_C0,
                          tail_start=(C // W, sc_end, C))
    return _combine(parts_tc, parts_sc, B)
